# Initial kernel scaffold; baseline (speedup 1.0000x reference)
import jax
import jax.numpy as jnp
from jax.experimental import pallas as pl


def kernel(vocab_word_idx, batch_unique_word_idx, obfuscation_vocab_random_indices_shuffle, obfuscation_embedding_table):
    raise NotImplementedError("write your pallas kernel here")


# speedup vs baseline: 16.1522x; 16.1522x over previous
import functools

import jax
import jax.numpy as jnp
from jax import lax
from jax.experimental import pallas as pl
from jax.experimental.pallas import tpu as pltpu, tpu_sc as plsc

VOCAB = 100000
EMBED = 64
PAD_IDX = 0
B, L = 16384, 50
N = B * L

NC, NS, LANES = 2, 16, 16
NW = NC * NS
PER_W = N // NW
CHUNK = 160
ITERS = PER_W // CHUNK
GROUPS = CHUNK // LANES


def _body(vocab_hbm, uniq_hbm, shuf_hbm, table_hbm, out_hbm,
          shuf_v, ui_v0, ui_v1, vi_v0, vi_v1, ridx_v0, ridx_v1,
          mask_v0, mask_v1, rows_v0, rows_v1, gsem0, gsem1):
    wid = lax.axis_index("s") * NC + lax.axis_index("c")
    base_w = wid * PER_W

    ui_v = (ui_v0, ui_v1)
    vi_v = (vi_v0, vi_v1)
    ridx_v = (ridx_v0, ridx_v1)
    mask_v = (mask_v0, mask_v1)
    rows_v = (rows_v0, rows_v1)

    pltpu.sync_copy(shuf_hbm, shuf_v)

    def stage_indices(i, buf):
        base = base_w + i * CHUNK
        pltpu.sync_copy(uniq_hbm.at[pl.ds(base, CHUNK)], ui_v[buf])
        pltpu.sync_copy(vocab_hbm.at[pl.ds(base, CHUNK)], vi_v[buf])

        def grp(g, pads):
            u = ui_v[buf][pl.ds(g * LANES, LANES)]
            ridx_v[buf][pl.ds(g * LANES, LANES)] = plsc.load_gather(
                shuf_v, [u])
            is_pad = vi_v[buf][pl.ds(g * LANES, LANES)] == PAD_IDX
            mask_v[buf][pl.ds(g * LANES, LANES)] = jnp.where(is_pad, 0.0, 1.0)
            return pads + lax.reduce_sum(
                jnp.where(is_pad, 1, 0), axes=(0,))

        return lax.fori_loop(0, GROUPS, grp, jnp.int32(0))

    def start_gather(buf, sem):
        pltpu.async_copy(table_hbm.at[ridx_v[buf]], rows_v[buf], sem)

    def wait_gather(buf, sem):
        pltpu.make_async_copy(table_hbm.at[ridx_v[buf]], rows_v[buf],
                              sem).wait()

    def finish_chunk(i, buf, pads):
        @pl.when(pads > 0)
        def _fixup():
            def rowfix(r, _):
                m = plsc.load_gather(mask_v[buf],
                                     [jnp.full((LANES,), r, jnp.int32)])
                for j in range(EMBED // LANES):
                    sl = pl.ds(j * LANES, LANES)
                    rows_v[buf][r, sl] = rows_v[buf][r, sl] * m
                return 0
            lax.fori_loop(0, CHUNK, rowfix, 0)

        base = base_w + i * CHUNK
        pltpu.sync_copy(rows_v[buf], out_hbm.at[pl.ds(base, CHUNK)])

    pads0 = stage_indices(0, 0)
    start_gather(0, gsem0)

    def pair(k, pads0):
        pads1 = stage_indices(2 * k + 1, 1)
        start_gather(1, gsem1)
        wait_gather(0, gsem0)
        finish_chunk(2 * k, 0, pads0)
        pads0n = stage_indices(2 * k + 2, 0)
        start_gather(0, gsem0)
        wait_gather(1, gsem1)
        finish_chunk(2 * k + 1, 1, pads1)
        return pads0n

    pads0 = lax.fori_loop(0, ITERS // 2 - 1, pair, pads0)
    pads1 = stage_indices(ITERS - 1, 1)
    start_gather(1, gsem1)
    wait_gather(0, gsem0)
    finish_chunk(ITERS - 2, 0, pads0)
    wait_gather(1, gsem1)
    finish_chunk(ITERS - 1, 1, pads1)


@jax.jit
def _run(vocab_flat, uniq_flat, shuffle, table):
    mesh = plsc.VectorSubcoreMesh(core_axis_name="c", subcore_axis_name="s")
    f = pl.kernel(
        _body,
        out_type=jax.ShapeDtypeStruct((N, EMBED), jnp.float32),
        mesh=mesh,
        compiler_params=pltpu.CompilerParams(needs_layout_passes=False,
                                             use_tc_tiling_on_sc=False),
        scratch_types=[
            pltpu.VMEM((VOCAB,), jnp.int32),
            pltpu.VMEM((CHUNK,), jnp.int32),
            pltpu.VMEM((CHUNK,), jnp.int32),
            pltpu.VMEM((CHUNK,), jnp.int32),
            pltpu.VMEM((CHUNK,), jnp.int32),
            pltpu.VMEM((CHUNK,), jnp.int32),
            pltpu.VMEM((CHUNK,), jnp.int32),
            pltpu.VMEM((CHUNK,), jnp.float32),
            pltpu.VMEM((CHUNK,), jnp.float32),
            pltpu.VMEM((CHUNK, EMBED), jnp.float32),
            pltpu.VMEM((CHUNK, EMBED), jnp.float32),
            pltpu.SemaphoreType.DMA,
            pltpu.SemaphoreType.DMA,
        ],
    )
    return f(vocab_flat, uniq_flat, shuffle, table)


def kernel(vocab_word_idx, batch_unique_word_idx,
           obfuscation_vocab_random_indices_shuffle,
           obfuscation_embedding_table):
    out = _run(vocab_word_idx.reshape(N),
               batch_unique_word_idx.reshape(N),
               obfuscation_vocab_random_indices_shuffle,
               obfuscation_embedding_table)
    return out.reshape(B, L, EMBED)


# speedup vs baseline: 17.2774x; 1.0697x over previous
import functools

import jax
import jax.numpy as jnp
from jax import lax
from jax.experimental import pallas as pl
from jax.experimental.pallas import tpu as pltpu, tpu_sc as plsc

VOCAB = 100000
EMBED = 64
PAD_IDX = 0
B, L = 16384, 50
N = B * L

NC, NS, LANES = 2, 16, 16
NW = NC * NS
PER_W = N // NW
CHUNK = 160
ITERS = PER_W // CHUNK
GROUPS = CHUNK // LANES


def _body(vocab_hbm, uniq_hbm, shuf_hbm, table_hbm, out_hbm,
          shuf_v, ui_v0, ui_v1, vi_v0, vi_v1, ridx_v0, ridx_v1,
          mask_v0, mask_v1, rows_v0, rows_v1,
          gsem0, gsem1, isem0, isem1, wsem0, wsem1):
    wid = lax.axis_index("s") * NC + lax.axis_index("c")
    base_w = wid * PER_W

    ui_v = (ui_v0, ui_v1)
    vi_v = (vi_v0, vi_v1)
    ridx_v = (ridx_v0, ridx_v1)
    mask_v = (mask_v0, mask_v1)
    rows_v = (rows_v0, rows_v1)
    gsem = (gsem0, gsem1)
    isem = (isem0, isem1)
    wsem = (wsem0, wsem1)

    pltpu.sync_copy(shuf_hbm, shuf_v)

    def prefetch_idx(i, s):
        base = base_w + i * CHUNK
        pltpu.async_copy(uniq_hbm.at[pl.ds(base, CHUNK)], ui_v[s], isem[s])
        pltpu.async_copy(vocab_hbm.at[pl.ds(base, CHUNK)], vi_v[s], isem[s])

    def compute(i, s):
        base = base_w + i * CHUNK
        pltpu.make_async_copy(uniq_hbm.at[pl.ds(base, CHUNK)], ui_v[s],
                              isem[s]).wait()
        pltpu.make_async_copy(vocab_hbm.at[pl.ds(base, CHUNK)], vi_v[s],
                              isem[s]).wait()

        def grp(g, acc):
            u = ui_v[s][pl.ds(g * LANES, LANES)]
            ridx_v[s][pl.ds(g * LANES, LANES)] = plsc.load_gather(shuf_v, [u])
            is_pad = vi_v[s][pl.ds(g * LANES, LANES)] == PAD_IDX
            mask_v[s][pl.ds(g * LANES, LANES)] = jnp.where(is_pad, 0.0, 1.0)
            return acc + jnp.where(is_pad, 1, 0)

        acc = lax.fori_loop(0, GROUPS, grp, jnp.zeros((LANES,), jnp.int32))
        return lax.reduce_sum(acc, axes=(0,))

    def start_gather(s):
        pltpu.async_copy(table_hbm.at[ridx_v[s]], rows_v[s], gsem[s])

    def wait_gather(s):
        pltpu.make_async_copy(table_hbm.at[ridx_v[s]], rows_v[s],
                              gsem[s]).wait()

    def fixup_and_wb(i, s, pads):
        @pl.when(pads > 0)
        def _fixup():
            def rowfix(r, _):
                m = plsc.load_gather(mask_v[s],
                                     [jnp.full((LANES,), r, jnp.int32)])
                for j in range(EMBED // LANES):
                    sl = pl.ds(j * LANES, LANES)
                    rows_v[s][r, sl] = rows_v[s][r, sl] * m
                return 0
            lax.fori_loop(0, CHUNK, rowfix, 0)

        base = base_w + i * CHUNK
        pltpu.async_copy(rows_v[s], out_hbm.at[pl.ds(base, CHUNK)], wsem[s])

    def wait_wb(i, s):
        base = base_w + i * CHUNK
        pltpu.make_async_copy(rows_v[s], out_hbm.at[pl.ds(base, CHUNK)],
                              wsem[s]).wait()

    prefetch_idx(0, 0)
    pads_a = compute(0, 0)
    start_gather(0)
    prefetch_idx(1, 1)

    pads_b = compute(1, 1)
    prefetch_idx(2, 0)
    wait_gather(0)
    fixup_and_wb(0, 0, pads_a)
    start_gather(1)
    pads_a = compute(2, 0)
    prefetch_idx(3, 1)
    wait_gather(1)
    fixup_and_wb(1, 1, pads_b)
    wait_wb(0, 0)
    start_gather(0)

    def pair(k, pads_a):
        a = 2 * k
        b = a + 1
        pads_b = compute(b, 1)
        prefetch_idx(a + 2, 0)
        wait_gather(0)
        fixup_and_wb(a, 0, pads_a)
        wait_wb(b - 2, 1)
        start_gather(1)
        pads_a2 = compute(a + 2, 0)
        prefetch_idx(b + 2, 1)
        wait_gather(1)
        fixup_and_wb(b, 1, pads_b)
        wait_wb(a, 0)
        start_gather(0)
        return pads_a2

    pads_a = lax.fori_loop(1, ITERS // 2 - 1, pair, pads_a)
    last = ITERS - 1
    pads_b = compute(last, 1)
    wait_gather(0)
    fixup_and_wb(last - 1, 0, pads_a)
    wait_wb(last - 2, 1)
    start_gather(1)
    wait_gather(1)
    fixup_and_wb(last, 1, pads_b)
    wait_wb(last - 1, 0)
    wait_wb(last, 1)


@jax.jit
def _run(vocab_flat, uniq_flat, shuffle, table):
    mesh = plsc.VectorSubcoreMesh(core_axis_name="c", subcore_axis_name="s")
    f = pl.kernel(
        _body,
        out_type=jax.ShapeDtypeStruct((N, EMBED), jnp.float32),
        mesh=mesh,
        compiler_params=pltpu.CompilerParams(needs_layout_passes=False,
                                             use_tc_tiling_on_sc=False),
        scratch_types=[
            pltpu.VMEM((VOCAB,), jnp.int32),
            pltpu.VMEM((CHUNK,), jnp.int32),
            pltpu.VMEM((CHUNK,), jnp.int32),
            pltpu.VMEM((CHUNK,), jnp.int32),
            pltpu.VMEM((CHUNK,), jnp.int32),
            pltpu.VMEM((CHUNK,), jnp.int32),
            pltpu.VMEM((CHUNK,), jnp.int32),
            pltpu.VMEM((CHUNK,), jnp.float32),
            pltpu.VMEM((CHUNK,), jnp.float32),
            pltpu.VMEM((CHUNK, EMBED), jnp.float32),
            pltpu.VMEM((CHUNK, EMBED), jnp.float32),
            pltpu.SemaphoreType.DMA,
            pltpu.SemaphoreType.DMA,
            pltpu.SemaphoreType.DMA,
            pltpu.SemaphoreType.DMA,
            pltpu.SemaphoreType.DMA,
            pltpu.SemaphoreType.DMA,
        ],
    )
    return f(vocab_flat, uniq_flat, shuffle, table)


def kernel(vocab_word_idx, batch_unique_word_idx,
           obfuscation_vocab_random_indices_shuffle,
           obfuscation_embedding_table):
    out = _run(vocab_word_idx.reshape(N),
               batch_unique_word_idx.reshape(N),
               obfuscation_vocab_random_indices_shuffle,
               obfuscation_embedding_table)
    return out.reshape(B, L, EMBED)
